# Initial kernel scaffold; baseline (speedup 1.0000x reference)
import jax
import jax.numpy as jnp
from jax.experimental import pallas as pl


def kernel(hidden_states, w13_stacked, w2_stacked, gate_w, shared_expert_gate_w, shared_gate_up_w, shared_down_w):
    raise NotImplementedError("write your pallas kernel here")


# speedup vs baseline: 1.5209x; 1.5209x over previous
import functools

import jax
import jax.numpy as jnp
from jax.experimental import pallas as pl
from jax.experimental.pallas import tpu as pltpu


def _silu(x):
    return x * jax.nn.sigmoid(x)


def _moe_body(x_ref, gate_ref, sgw_ref, w13_ref, w2_ref, wg_ref, wu_ref,
              wd_ref, out_ref, *, TM, E, I):
    e = pl.program_id(0)
    mt = pl.program_id(1)

    xb = x_ref[pl.ds(mt * TM, TM), :]
    xb16 = xb.astype(jnp.bfloat16)

    logits = jax.lax.dot_general(
        xb16, gate_ref[...].astype(jnp.bfloat16), (((1,), (1,)), ((), ())),
        preferred_element_type=jnp.float32)
    w = jax.nn.softmax(logits, axis=-1)
    iota = jax.lax.broadcasted_iota(jnp.int32, w.shape, 1)
    m1 = jnp.max(w, axis=-1, keepdims=True)
    i1 = jnp.min(jnp.where(w == m1, iota, E), axis=-1, keepdims=True)
    wm = jnp.where(iota == i1, -1.0, w)
    m2 = jnp.max(wm, axis=-1, keepdims=True)
    i2 = jnp.min(jnp.where(wm == m2, iota, E), axis=-1, keepdims=True)
    combine = jnp.where((iota == i1) | (iota == i2), w, 0.0)
    ce = jnp.sum(jnp.where(iota == e, combine, 0.0), axis=-1,
                 keepdims=True)

    w13 = w13_ref[0].astype(jnp.bfloat16)
    h = jax.lax.dot_general(xb16, w13, (((1,), (1,)), ((), ())),
                            preferred_element_type=jnp.float32)
    act = (_silu(h[:, :I]) * h[:, I:]).astype(jnp.bfloat16)
    w2 = w2_ref[0].astype(jnp.bfloat16)
    eo = jax.lax.dot_general(act, w2, (((1,), (1,)), ((), ())),
                             preferred_element_type=jnp.float32)
    acc = ce * eo

    wg = wg_ref[...].astype(jnp.bfloat16)
    wu = wu_ref[...].astype(jnp.bfloat16)
    gs = jax.lax.dot_general(xb16, wg, (((1,), (1,)), ((), ())),
                             preferred_element_type=jnp.float32)
    us = jax.lax.dot_general(xb16, wu, (((1,), (1,)), ((), ())),
                             preferred_element_type=jnp.float32)
    sa = (_silu(gs) * us).astype(jnp.bfloat16)
    wd = wd_ref[...].astype(jnp.bfloat16)
    so = jax.lax.dot_general(sa, wd, (((1,), (1,)), ((), ())),
                             preferred_element_type=jnp.float32)
    sg = jax.lax.dot_general(
        xb, sgw_ref[...], (((1,), (1,)), ((), ())),
        precision=jax.lax.Precision.HIGHEST,
        preferred_element_type=jnp.float32)
    acc = acc + so * jax.nn.sigmoid(sg)

    @pl.when(e == 0)
    def _init():
        out_ref[pl.ds(mt * TM, TM), :] = acc

    @pl.when(e != 0)
    def _accum():
        out_ref[pl.ds(mt * TM, TM), :] = out_ref[pl.ds(mt * TM, TM), :] + acc


def kernel(hidden_states, w13_stacked, w2_stacked, gate_w,
           shared_expert_gate_w, shared_gate_up_w, shared_down_w):
    orig_shape = hidden_states.shape
    H = orig_shape[-1]
    x = hidden_states.reshape(-1, H)
    M = x.shape[0]
    E, twoI, _ = w13_stacked.shape
    I = twoI // 2
    S = shared_down_w.shape[1]
    TS = S // E
    TM = min(256, M)
    MT = M // TM

    grid = (E, MT)
    out = pl.pallas_call(
        functools.partial(_moe_body, TM=TM, E=E, I=I),
        grid=grid,
        in_specs=[
            pl.BlockSpec((M, H), lambda e, mt: (0, 0)),
            pl.BlockSpec((E, H), lambda e, mt: (0, 0)),
            pl.BlockSpec((1, H), lambda e, mt: (0, 0)),
            pl.BlockSpec((1, twoI, H), lambda e, mt: (e, 0, 0)),
            pl.BlockSpec((1, H, I), lambda e, mt: (e, 0, 0)),
            pl.BlockSpec((TS, H), lambda e, mt: (e, 0)),
            pl.BlockSpec((TS, H), lambda e, mt: (e + E, 0)),
            pl.BlockSpec((H, TS), lambda e, mt: (0, e)),
        ],
        out_specs=pl.BlockSpec((M, H), lambda e, mt: (0, 0)),
        out_shape=jax.ShapeDtypeStruct((M, H), jnp.float32),
        compiler_params=pltpu.CompilerParams(
            dimension_semantics=("arbitrary", "arbitrary")),
    )(x, gate_w, shared_expert_gate_w, w13_stacked, w2_stacked,
      shared_gate_up_w, shared_gate_up_w, shared_down_w)
    return out.reshape(orig_shape)
